# Initial kernel scaffold; baseline (speedup 1.0000x reference)
import jax
import jax.numpy as jnp
from jax.experimental import pallas as pl


def kernel(x_sem, edge_index, batch_index, gcn_W, gcn_b, pt_W, pt_b, ps_W, ps_b, s1_W, s1_b, s2_W, s2_b, cls_W, cls_b):
    raise NotImplementedError("write your pallas kernel here")


# speedup vs baseline: 18.8088x; 18.8088x over previous
import functools

import jax
import jax.numpy as jnp
from jax import lax
from jax.experimental import pallas as pl
from jax.experimental.pallas import tpu as pltpu
from jax.experimental.pallas import tpu_sc as plsc

N = 10000
NPAD = 10240
E = 320000
CHUNK = 128
NC = 2
NS = 16
NW = NC * NS
EPAD = ((E + NW * CHUNK - 1) // (NW * CHUNK)) * (NW * CHUNK)
EPW = EPAD // NW
NCHUNK = EPW // CHUNK
RPT = NPAD // NS
HID = 64
DEGW = 16

_HIGH = jax.lax.Precision.HIGHEST


def _bdot(a, b):
    return jnp.dot(a.astype(jnp.bfloat16), b.astype(jnp.bfloat16),
                   preferred_element_type=jnp.float32)


def _sc_degree_body(dst_hbm, ones_hbm, zeros_hbm, out_hbm, idx_v, ones_v, acc_sh):
    c = lax.axis_index("c")
    s = lax.axis_index("s")
    w = c * NS + s
    pltpu.sync_copy(ones_hbm, ones_v)
    pltpu.sync_copy(zeros_hbm, acc_sh.at[pl.ds(s * RPT, RPT)])
    plsc.subcore_barrier()
    base = w * jnp.int32(EPW)

    def chunk(j, carry):
        off = pl.multiple_of(base + j * jnp.int32(CHUNK), CHUNK)
        pltpu.sync_copy(dst_hbm.at[pl.ds(off, CHUNK)], idx_v)
        pltpu.sync_copy(ones_v, acc_sh.at[idx_v], add=True)
        return carry

    lax.fori_loop(jnp.int32(0), jnp.int32(NCHUNK), chunk, jnp.int32(0))
    plsc.subcore_barrier()
    pltpu.sync_copy(acc_sh.at[pl.ds(s * RPT, RPT)],
                    out_hbm.at[c, pl.ds(s * RPT, RPT)])


def _sc_aggregate_body(src_hbm, dst_hbm, y_hbm, zeros_hbm, out_hbm,
                       src_v, dst_v, rows_v, sem, acc_sh):
    c = lax.axis_index("c")
    s = lax.axis_index("s")
    w = c * NS + s
    pltpu.sync_copy(zeros_hbm, acc_sh.at[pl.ds(s * RPT, RPT)])
    plsc.subcore_barrier()
    base = w * jnp.int32(EPW)

    def chunk(j, carry):
        off = pl.multiple_of(base + j * jnp.int32(CHUNK), CHUNK)
        pltpu.sync_copy(src_hbm.at[pl.ds(off, CHUNK)], src_v)
        pltpu.sync_copy(dst_hbm.at[pl.ds(off, CHUNK)], dst_v)
        pltpu.async_copy(y_hbm.at[src_v], rows_v, sem).wait()
        pltpu.sync_copy(rows_v, acc_sh.at[dst_v], add=True)
        return carry

    lax.fori_loop(jnp.int32(0), jnp.int32(NCHUNK), chunk, jnp.int32(0))
    plsc.subcore_barrier()
    pltpu.sync_copy(acc_sh.at[pl.ds(s * RPT, RPT)],
                    out_hbm.at[c, pl.ds(s * RPT, RPT)])


@functools.lru_cache(maxsize=None)
def _sc_kernels():
    mesh = plsc.VectorSubcoreMesh(core_axis_name="c", subcore_axis_name="s")
    params = pltpu.CompilerParams(use_tc_tiling_on_sc=False)
    degree = functools.partial(
        pl.kernel,
        mesh=mesh,
        compiler_params=params,
        out_type=jax.ShapeDtypeStruct((NC, NPAD, DEGW), jnp.float32),
        scratch_types=[
            pltpu.VMEM((CHUNK,), jnp.int32),
            pltpu.VMEM((CHUNK, DEGW), jnp.float32),
            pltpu.VMEM_SHARED((NPAD, DEGW), jnp.float32),
        ],
    )(_sc_degree_body)
    aggregate = functools.partial(
        pl.kernel,
        mesh=mesh,
        compiler_params=params,
        out_type=jax.ShapeDtypeStruct((NC, NPAD, HID), jnp.float32),
        scratch_types=[
            pltpu.VMEM((CHUNK,), jnp.int32),
            pltpu.VMEM((CHUNK,), jnp.int32),
            pltpu.VMEM((CHUNK, HID), jnp.float32),
            pltpu.SemaphoreType.DMA,
            pltpu.VMEM_SHARED((NPAD, HID), jnp.float32),
        ],
    )(_sc_aggregate_body)
    return degree, aggregate


def _tc_prescale_body(x_ref, w_ref, degp_ref, y_ref, dinv_ref):
    xw = _bdot(x_ref[...], w_ref[...])
    degp = degp_ref[...]
    deg = degp[0, :, 0] + degp[1, :, 0] + 1.0
    dinv = lax.rsqrt(deg)
    dinv_ref[...] = dinv
    y_ref[0:N, :] = dinv[0:N][:, None] * xw
    y_ref[N:NPAD, :] = jnp.zeros((NPAD - N, HID), jnp.float32)


def _tc_prescale(x, gcn_w, degp):
    return pl.pallas_call(
        _tc_prescale_body,
        out_shape=(jax.ShapeDtypeStruct((NPAD, HID), jnp.float32),
                   jax.ShapeDtypeStruct((NPAD,), jnp.float32)),
    )(x, gcn_w, degp)


def _tc_proj_body(aggp_ref, y_ref, dinv_ref, gcn_b_ref, h_ref):
    agg = aggp_ref[0, 0:N, :] + aggp_ref[1, 0:N, :] + y_ref[0:N, :]
    h = dinv_ref[0:N][:, None] * agg + gcn_b_ref[...]
    h_ref[...] = jnp.maximum(h, 0.0)


def _tc_proj(aggp, y, dinv, gcn_b):
    return pl.pallas_call(
        _tc_proj_body,
        out_shape=jax.ShapeDtypeStruct((N, HID), jnp.float32),
    )(aggp, y, dinv, gcn_b)


def _tc_score_body(h_ref, x_ref, pt_w_ref, pt_b_ref, ps_w_ref, ps_b_ref,
                   s1_w_ref, s1_b_ref, s2_w_ref, s2_b_ref,
                   z_ref, si_ref):
    z_topo = _bdot(h_ref[...], pt_w_ref[...]) + pt_b_ref[...]
    z_sem = _bdot(x_ref[...], ps_w_ref[...]) + ps_b_ref[...]
    z_ref[...] = jnp.concatenate([z_topo, z_sem], axis=-1)
    diff = jnp.abs(z_topo - z_sem)
    cf = jnp.concatenate([z_topo, z_sem, diff], axis=-1)
    hs = _bdot(cf, s1_w_ref[...]) + s1_b_ref[...]
    hs = jnp.where(hs >= 0, hs, 0.01 * hs)
    s_i = _bdot(hs, s2_w_ref[...]) + s2_b_ref[...]
    si_ref[...] = s_i[:, 0]


def _tc_score(h_topo, x, pt_w, pt_b, ps_w, ps_b, s1_w, s1_b, s2_w, s2_b):
    return pl.pallas_call(
        _tc_score_body,
        out_shape=(jax.ShapeDtypeStruct((N, 64), jnp.float32),
                   jax.ShapeDtypeStruct((N,), jnp.float32)),
    )(h_topo, x, pt_w, pt_b, ps_w, ps_b, s1_w, s1_b, s2_w, s2_b)


def _tc_pool_body(si_ref, batch_ref, h_ref, cls_w_ref, cls_b_ref,
                  logits_ref, alpha_ref):
    s_i = si_ref[...][:, None]
    gids = lax.broadcasted_iota(jnp.int32, (1, 64), 1)
    mask = (batch_ref[...][:, None] == gids).astype(jnp.float32)
    masked = jnp.where(mask > 0, s_i, jnp.float32(-1e30))
    seg_max = jnp.max(masked, axis=0, keepdims=True)
    seg_max = jnp.where(seg_max > -1e29, seg_max, 0.0)
    node_max = jnp.sum(mask * seg_max, axis=1, keepdims=True)
    e = jnp.exp(s_i - node_max)
    seg_sum = lax.dot_general(mask, e, (((0,), (0,)), ((), ())),
                              precision=_HIGH)
    node_sum = jnp.dot(mask, seg_sum,
                       preferred_element_type=jnp.float32, precision=_HIGH)
    alpha = e / (node_sum + 1e-16)
    wh = h_ref[...] * alpha
    z_graph = lax.dot_general(mask, wh, (((0,), (0,)), ((), ())),
                              precision=_HIGH)
    logits = _bdot(z_graph, cls_w_ref[...]) + cls_b_ref[...]
    logits_ref[...] = logits
    alpha_ref[...] = alpha


def _tc_pool(s_i, batch1, h_topo, cls_w, cls_b):
    return pl.pallas_call(
        _tc_pool_body,
        out_shape=(jax.ShapeDtypeStruct((64, 1), jnp.float32),
                   jax.ShapeDtypeStruct((N, 1), jnp.float32)),
    )(s_i, batch1, h_topo, cls_w, cls_b)


def kernel(x_sem, edge_index, batch_index, gcn_W, gcn_b, pt_W, pt_b,
           ps_W, ps_b, s1_W, s1_b, s2_W, s2_b, cls_W, cls_b):
    f32 = jnp.float32
    src = edge_index[0].astype(jnp.int32)
    dst = edge_index[1].astype(jnp.int32)
    pad = jnp.full((EPAD - E,), NPAD - 1, jnp.int32)
    src_p = jnp.concatenate([src, pad])
    dst_p = jnp.concatenate([dst, pad])
    batch1 = batch_index.astype(jnp.int32)

    ones_deg = jnp.ones((CHUNK, DEGW), f32)
    zeros_deg = jnp.zeros((RPT, DEGW), f32)
    zeros_agg = jnp.zeros((RPT, HID), f32)

    sc_degree, sc_aggregate = _sc_kernels()
    degp = sc_degree(dst_p, ones_deg, zeros_deg)
    y, dinv = _tc_prescale(x_sem.astype(f32), gcn_W.astype(f32), degp)
    aggp = sc_aggregate(src_p, dst_p, y, zeros_agg)

    h_topo = _tc_proj(aggp, y, dinv, gcn_b.astype(f32).reshape(1, HID))
    z_both, s_i = _tc_score(
        h_topo, x_sem.astype(f32),
        pt_W.astype(f32), pt_b.astype(f32).reshape(1, 32),
        ps_W.astype(f32), ps_b.astype(f32).reshape(1, 32),
        s1_W.astype(f32), s1_b.astype(f32).reshape(1, 16),
        s2_W.astype(f32), s2_b.astype(f32).reshape(1, 1))
    logits2, alpha = _tc_pool(s_i, batch1, h_topo,
                              cls_W.astype(f32),
                              cls_b.astype(f32).reshape(1, 1))
    return (logits2.reshape(64), alpha, z_both[:, 0:32], z_both[:, 32:64])


# speedup vs baseline: 19.8316x; 1.0544x over previous
import functools

import jax
import jax.numpy as jnp
from jax import lax
from jax.experimental import pallas as pl
from jax.experimental.pallas import tpu as pltpu
from jax.experimental.pallas import tpu_sc as plsc

N = 10000
NPAD = 10240
E = 320000
CHUNK = 128
NC = 2
NS = 16
NW = NC * NS
KB = 8
EPAD = ((E + NW * CHUNK * KB - 1) // (NW * CHUNK * KB)) * (NW * CHUNK * KB)
EPW = EPAD // NW
NCHUNK = EPW // CHUNK
ITERS = NCHUNK // KB
RPT = NPAD // NS
HID = 64
DEGW = 16

_HIGH = jax.lax.Precision.HIGHEST


def _bdot(a, b):
    return jnp.dot(a.astype(jnp.bfloat16), b.astype(jnp.bfloat16),
                   preferred_element_type=jnp.float32)


def _sc_degree_body(dst_hbm, ones_hbm, zeros_hbm, out_hbm, idx_v, ones_v,
                    sem_s, acc_sh):
    c = lax.axis_index("c")
    s = lax.axis_index("s")
    w = c * NS + s
    pltpu.sync_copy(ones_hbm, ones_v)
    pltpu.sync_copy(zeros_hbm, acc_sh.at[pl.ds(s * RPT, RPT)])
    plsc.subcore_barrier()
    base = w * jnp.int32(NCHUNK)

    def step(j, carry):
        row = base + j * jnp.int32(KB)
        pltpu.sync_copy(dst_hbm.at[pl.ds(row, KB)], idx_v)
        descs = [pltpu.async_copy(ones_v, acc_sh.at[idx_v.at[jnp.int32(k)]], sem_s,
                                  add=True) for k in range(KB)]
        for d in descs:
            d.wait()
        return carry

    lax.fori_loop(jnp.int32(0), jnp.int32(ITERS), step, jnp.int32(0))
    plsc.subcore_barrier()
    pltpu.sync_copy(acc_sh.at[pl.ds(s * RPT, RPT)],
                    out_hbm.at[c, pl.ds(s * RPT, RPT)])


def _sc_aggregate_body(src_hbm, dst_hbm, y_hbm, zeros_hbm, out_hbm,
                       src_v, dst_v, rows_v, sem_g, sem_s, acc_sh):
    c = lax.axis_index("c")
    s = lax.axis_index("s")
    w = c * NS + s
    pltpu.sync_copy(zeros_hbm, acc_sh.at[pl.ds(s * RPT, RPT)])
    plsc.subcore_barrier()
    base = w * jnp.int32(NCHUNK)

    def step(j, carry):
        row = base + j * jnp.int32(KB)
        pltpu.sync_copy(src_hbm.at[pl.ds(row, KB)], src_v)
        pltpu.sync_copy(dst_hbm.at[pl.ds(row, KB)], dst_v)
        gds = [pltpu.async_copy(y_hbm.at[src_v.at[jnp.int32(k)]], rows_v.at[jnp.int32(k)], sem_g)
               for k in range(KB)]
        for d in gds:
            d.wait()
        sds = [pltpu.async_copy(rows_v.at[jnp.int32(k)], acc_sh.at[dst_v.at[jnp.int32(k)]], sem_s,
                                add=True) for k in range(KB)]
        for d in sds:
            d.wait()
        return carry

    lax.fori_loop(jnp.int32(0), jnp.int32(ITERS), step, jnp.int32(0))
    plsc.subcore_barrier()
    pltpu.sync_copy(acc_sh.at[pl.ds(s * RPT, RPT)],
                    out_hbm.at[c, pl.ds(s * RPT, RPT)])


@functools.lru_cache(maxsize=None)
def _sc_kernels():
    mesh = plsc.VectorSubcoreMesh(core_axis_name="c", subcore_axis_name="s")
    params = pltpu.CompilerParams(use_tc_tiling_on_sc=False)
    degree = functools.partial(
        pl.kernel,
        mesh=mesh,
        compiler_params=params,
        out_type=jax.ShapeDtypeStruct((NC, NPAD, DEGW), jnp.float32),
        scratch_types=[
            pltpu.VMEM((KB, CHUNK), jnp.int32),
            pltpu.VMEM((CHUNK, DEGW), jnp.float32),
            pltpu.SemaphoreType.DMA,
            pltpu.VMEM_SHARED((NPAD, DEGW), jnp.float32),
        ],
    )(_sc_degree_body)
    aggregate = functools.partial(
        pl.kernel,
        mesh=mesh,
        compiler_params=params,
        out_type=jax.ShapeDtypeStruct((NC, NPAD, HID), jnp.float32),
        scratch_types=[
            pltpu.VMEM((KB, CHUNK), jnp.int32),
            pltpu.VMEM((KB, CHUNK), jnp.int32),
            pltpu.VMEM((KB, CHUNK, HID), jnp.float32),
            pltpu.SemaphoreType.DMA,
            pltpu.SemaphoreType.DMA,
            pltpu.VMEM_SHARED((NPAD, HID), jnp.float32),
        ],
    )(_sc_aggregate_body)
    return degree, aggregate


def _tc_prescale_body(x_ref, w_ref, degp_ref, y_ref, dinv_ref):
    xw = _bdot(x_ref[...], w_ref[...])
    degp = degp_ref[...]
    deg = degp[0, :, 0] + degp[1, :, 0] + 1.0
    dinv = lax.rsqrt(deg)
    dinv_ref[...] = dinv
    y_ref[0:N, :] = dinv[0:N][:, None] * xw
    y_ref[N:NPAD, :] = jnp.zeros((NPAD - N, HID), jnp.float32)


def _tc_prescale(x, gcn_w, degp):
    return pl.pallas_call(
        _tc_prescale_body,
        out_shape=(jax.ShapeDtypeStruct((NPAD, HID), jnp.float32),
                   jax.ShapeDtypeStruct((NPAD,), jnp.float32)),
    )(x, gcn_w, degp)


def _tc_proj_body(aggp_ref, y_ref, dinv_ref, gcn_b_ref, h_ref):
    agg = aggp_ref[0, 0:N, :] + aggp_ref[1, 0:N, :] + y_ref[0:N, :]
    h = dinv_ref[0:N][:, None] * agg + gcn_b_ref[...]
    h_ref[...] = jnp.maximum(h, 0.0)


def _tc_proj(aggp, y, dinv, gcn_b):
    return pl.pallas_call(
        _tc_proj_body,
        out_shape=jax.ShapeDtypeStruct((N, HID), jnp.float32),
    )(aggp, y, dinv, gcn_b)


def _tc_score_body(h_ref, x_ref, pt_w_ref, pt_b_ref, ps_w_ref, ps_b_ref,
                   s1_w_ref, s1_b_ref, s2_w_ref, s2_b_ref,
                   z_ref, si_ref):
    z_topo = _bdot(h_ref[...], pt_w_ref[...]) + pt_b_ref[...]
    z_sem = _bdot(x_ref[...], ps_w_ref[...]) + ps_b_ref[...]
    z_ref[...] = jnp.concatenate([z_topo, z_sem], axis=-1)
    diff = jnp.abs(z_topo - z_sem)
    cf = jnp.concatenate([z_topo, z_sem, diff], axis=-1)
    hs = _bdot(cf, s1_w_ref[...]) + s1_b_ref[...]
    hs = jnp.where(hs >= 0, hs, 0.01 * hs)
    s_i = _bdot(hs, s2_w_ref[...]) + s2_b_ref[...]
    si_ref[...] = s_i[:, 0]


def _tc_score(h_topo, x, pt_w, pt_b, ps_w, ps_b, s1_w, s1_b, s2_w, s2_b):
    return pl.pallas_call(
        _tc_score_body,
        out_shape=(jax.ShapeDtypeStruct((N, 64), jnp.float32),
                   jax.ShapeDtypeStruct((N,), jnp.float32)),
    )(h_topo, x, pt_w, pt_b, ps_w, ps_b, s1_w, s1_b, s2_w, s2_b)


def _tc_pool_body(si_ref, batch_ref, h_ref, cls_w_ref, cls_b_ref,
                  logits_ref, alpha_ref):
    s_i = si_ref[...][:, None]
    gids = lax.broadcasted_iota(jnp.int32, (1, 64), 1)
    mask = (batch_ref[...][:, None] == gids).astype(jnp.float32)
    masked = jnp.where(mask > 0, s_i, jnp.float32(-1e30))
    seg_max = jnp.max(masked, axis=0, keepdims=True)
    seg_max = jnp.where(seg_max > -1e29, seg_max, 0.0)
    node_max = jnp.sum(mask * seg_max, axis=1, keepdims=True)
    e = jnp.exp(s_i - node_max)
    seg_sum = lax.dot_general(mask, e, (((0,), (0,)), ((), ())),
                              precision=_HIGH)
    node_sum = jnp.dot(mask, seg_sum,
                       preferred_element_type=jnp.float32, precision=_HIGH)
    alpha = e / (node_sum + 1e-16)
    wh = h_ref[...] * alpha
    z_graph = lax.dot_general(mask, wh, (((0,), (0,)), ((), ())),
                              precision=_HIGH)
    logits = _bdot(z_graph, cls_w_ref[...]) + cls_b_ref[...]
    logits_ref[...] = logits
    alpha_ref[...] = alpha


def _tc_pool(s_i, batch1, h_topo, cls_w, cls_b):
    return pl.pallas_call(
        _tc_pool_body,
        out_shape=(jax.ShapeDtypeStruct((64, 1), jnp.float32),
                   jax.ShapeDtypeStruct((N, 1), jnp.float32)),
    )(s_i, batch1, h_topo, cls_w, cls_b)


def kernel(x_sem, edge_index, batch_index, gcn_W, gcn_b, pt_W, pt_b,
           ps_W, ps_b, s1_W, s1_b, s2_W, s2_b, cls_W, cls_b):
    f32 = jnp.float32
    src = edge_index[0].astype(jnp.int32)
    dst = edge_index[1].astype(jnp.int32)
    pad = jnp.full((EPAD - E,), NPAD - 1, jnp.int32)
    src_p = jnp.concatenate([src, pad]).reshape(EPAD // CHUNK, CHUNK)
    dst_p = jnp.concatenate([dst, pad]).reshape(EPAD // CHUNK, CHUNK)
    batch1 = batch_index.astype(jnp.int32)

    ones_deg = jnp.ones((CHUNK, DEGW), f32)
    zeros_deg = jnp.zeros((RPT, DEGW), f32)
    zeros_agg = jnp.zeros((RPT, HID), f32)

    sc_degree, sc_aggregate = _sc_kernels()
    degp = sc_degree(dst_p, ones_deg, zeros_deg)
    y, dinv = _tc_prescale(x_sem.astype(f32), gcn_W.astype(f32), degp)
    aggp = sc_aggregate(src_p, dst_p, y, zeros_agg)

    h_topo = _tc_proj(aggp, y, dinv, gcn_b.astype(f32).reshape(1, HID))
    z_both, s_i = _tc_score(
        h_topo, x_sem.astype(f32),
        pt_W.astype(f32), pt_b.astype(f32).reshape(1, 32),
        ps_W.astype(f32), ps_b.astype(f32).reshape(1, 32),
        s1_W.astype(f32), s1_b.astype(f32).reshape(1, 16),
        s2_W.astype(f32), s2_b.astype(f32).reshape(1, 1))
    logits2, alpha = _tc_pool(s_i, batch1, h_topo,
                              cls_W.astype(f32),
                              cls_b.astype(f32).reshape(1, 1))
    return (logits2.reshape(64), alpha, z_both[:, 0:32], z_both[:, 32:64])


# speedup vs baseline: 31.2535x; 1.5759x over previous
import functools

import jax
import jax.numpy as jnp
from jax import lax
from jax.experimental import pallas as pl
from jax.experimental.pallas import tpu as pltpu
from jax.experimental.pallas import tpu_sc as plsc

N = 10000
NPAD = 10240
E = 320000
CHUNK = 128
NC = 2
NS = 16
NW = NC * NS
KB = 8
EPAD = ((E + NW * CHUNK * KB - 1) // (NW * CHUNK * KB)) * (NW * CHUNK * KB)
EPW = EPAD // NW
NCHUNK = EPW // CHUNK
ITERS = NCHUNK // KB
RPT = NPAD // NS
HID = 64
HHID = HID // 2
NCHUNK2 = EPAD // CHUNK // NS
ITERS2 = NCHUNK2 // KB
DEGW = 16

_HIGH = jax.lax.Precision.HIGHEST


def _bdot(a, b):
    return jnp.dot(a.astype(jnp.bfloat16), b.astype(jnp.bfloat16),
                   preferred_element_type=jnp.float32)


def _sc_degree_body(dst_hbm, ones_hbm, zeros_hbm, out_hbm, idx_v, ones_v,
                    sem_s, acc_sh):
    c = lax.axis_index("c")
    s = lax.axis_index("s")
    w = c * NS + s
    pltpu.sync_copy(ones_hbm, ones_v)
    pltpu.sync_copy(zeros_hbm, acc_sh.at[pl.ds(s * RPT, RPT)])
    plsc.subcore_barrier()
    base = w * jnp.int32(NCHUNK)

    def step(j, carry):
        row = base + j * jnp.int32(KB)
        pltpu.sync_copy(dst_hbm.at[pl.ds(row, KB)], idx_v)
        descs = [pltpu.async_copy(ones_v, acc_sh.at[idx_v.at[jnp.int32(k)]], sem_s,
                                  add=True) for k in range(KB)]
        for d in descs:
            d.wait()
        return carry

    lax.fori_loop(jnp.int32(0), jnp.int32(ITERS), step, jnp.int32(0))
    plsc.subcore_barrier()
    pltpu.sync_copy(acc_sh.at[pl.ds(s * RPT, RPT)],
                    out_hbm.at[c, pl.ds(s * RPT, RPT)])


def _sc_aggregate_body(src_hbm, dst_hbm, y2_hbm, zeros_hbm, out_hbm,
                       src_v, dst_v, rows_v, sem_g, sem_s, acc_sh, y_sh):
    c = lax.axis_index("c")
    s = lax.axis_index("s")
    pltpu.sync_copy(zeros_hbm, acc_sh.at[pl.ds(s * RPT, RPT)])
    pltpu.sync_copy(y2_hbm.at[c, pl.ds(s * RPT, RPT)],
                    y_sh.at[pl.ds(s * RPT, RPT)])
    plsc.subcore_barrier()
    base = s * jnp.int32(NCHUNK2)

    def step(j, carry):
        row = base + j * jnp.int32(KB)
        pltpu.sync_copy(src_hbm.at[pl.ds(row, KB)], src_v)
        pltpu.sync_copy(dst_hbm.at[pl.ds(row, KB)], dst_v)
        gds = [pltpu.async_copy(y_sh.at[src_v.at[jnp.int32(k)]], rows_v.at[jnp.int32(k)], sem_g)
               for k in range(KB)]
        for d in gds:
            d.wait()
        sds = [pltpu.async_copy(rows_v.at[jnp.int32(k)], acc_sh.at[dst_v.at[jnp.int32(k)]], sem_s,
                                add=True) for k in range(KB)]
        for d in sds:
            d.wait()
        return carry

    lax.fori_loop(jnp.int32(0), jnp.int32(ITERS2), step, jnp.int32(0))
    plsc.subcore_barrier()
    pltpu.sync_copy(acc_sh.at[pl.ds(s * RPT, RPT)],
                    out_hbm.at[c, pl.ds(s * RPT, RPT)])


@functools.lru_cache(maxsize=None)
def _sc_kernels():
    mesh = plsc.VectorSubcoreMesh(core_axis_name="c", subcore_axis_name="s")
    params = pltpu.CompilerParams(use_tc_tiling_on_sc=False)
    degree = functools.partial(
        pl.kernel,
        mesh=mesh,
        compiler_params=params,
        out_type=jax.ShapeDtypeStruct((NC, NPAD, DEGW), jnp.float32),
        scratch_types=[
            pltpu.VMEM((KB, CHUNK), jnp.int32),
            pltpu.VMEM((CHUNK, DEGW), jnp.float32),
            pltpu.SemaphoreType.DMA,
            pltpu.VMEM_SHARED((NPAD, DEGW), jnp.float32),
        ],
    )(_sc_degree_body)
    aggregate = functools.partial(
        pl.kernel,
        mesh=mesh,
        compiler_params=params,
        out_type=jax.ShapeDtypeStruct((NC, NPAD, HHID), jnp.float32),
        scratch_types=[
            pltpu.VMEM((KB, CHUNK), jnp.int32),
            pltpu.VMEM((KB, CHUNK), jnp.int32),
            pltpu.VMEM((KB, CHUNK, HHID), jnp.float32),
            pltpu.SemaphoreType.DMA,
            pltpu.SemaphoreType.DMA,
            pltpu.VMEM_SHARED((NPAD, HHID), jnp.float32),
            pltpu.VMEM_SHARED((NPAD, HHID), jnp.float32),
        ],
    )(_sc_aggregate_body)
    return degree, aggregate


def _tc_prescale_body(x_ref, w_ref, degp_ref, y_ref, dinv_ref):
    xw = _bdot(x_ref[...], w_ref[...])
    degp = degp_ref[...]
    deg = degp[0, :, 0] + degp[1, :, 0] + 1.0
    dinv = lax.rsqrt(deg)
    dinv_ref[...] = dinv
    y = dinv[0:N][:, None] * xw
    y_ref[0, 0:N, :] = y[:, 0:HHID]
    y_ref[1, 0:N, :] = y[:, HHID:HID]
    y_ref[:, N:NPAD, :] = jnp.zeros((2, NPAD - N, HHID), jnp.float32)


def _tc_prescale(x, gcn_w, degp):
    return pl.pallas_call(
        _tc_prescale_body,
        out_shape=(jax.ShapeDtypeStruct((2, NPAD, HHID), jnp.float32),
                   jax.ShapeDtypeStruct((NPAD,), jnp.float32)),
    )(x, gcn_w, degp)


def _tc_proj_body(aggp_ref, y_ref, dinv_ref, gcn_b_ref, h_ref):
    agg = jnp.concatenate([aggp_ref[0, 0:N, :] + y_ref[0, 0:N, :],
                           aggp_ref[1, 0:N, :] + y_ref[1, 0:N, :]], axis=-1)
    h = dinv_ref[0:N][:, None] * agg + gcn_b_ref[...]
    h_ref[...] = jnp.maximum(h, 0.0)


def _tc_proj(aggp, y, dinv, gcn_b):
    return pl.pallas_call(
        _tc_proj_body,
        out_shape=jax.ShapeDtypeStruct((N, HID), jnp.float32),
    )(aggp, y, dinv, gcn_b)


def _tc_score_body(h_ref, x_ref, pt_w_ref, pt_b_ref, ps_w_ref, ps_b_ref,
                   s1_w_ref, s1_b_ref, s2_w_ref, s2_b_ref,
                   z_ref, si_ref):
    z_topo = _bdot(h_ref[...], pt_w_ref[...]) + pt_b_ref[...]
    z_sem = _bdot(x_ref[...], ps_w_ref[...]) + ps_b_ref[...]
    z_ref[...] = jnp.concatenate([z_topo, z_sem], axis=-1)
    diff = jnp.abs(z_topo - z_sem)
    cf = jnp.concatenate([z_topo, z_sem, diff], axis=-1)
    hs = _bdot(cf, s1_w_ref[...]) + s1_b_ref[...]
    hs = jnp.where(hs >= 0, hs, 0.01 * hs)
    s_i = _bdot(hs, s2_w_ref[...]) + s2_b_ref[...]
    si_ref[...] = s_i[:, 0]


def _tc_score(h_topo, x, pt_w, pt_b, ps_w, ps_b, s1_w, s1_b, s2_w, s2_b):
    return pl.pallas_call(
        _tc_score_body,
        out_shape=(jax.ShapeDtypeStruct((N, 64), jnp.float32),
                   jax.ShapeDtypeStruct((N,), jnp.float32)),
    )(h_topo, x, pt_w, pt_b, ps_w, ps_b, s1_w, s1_b, s2_w, s2_b)


def _tc_pool_body(si_ref, batch_ref, h_ref, cls_w_ref, cls_b_ref,
                  logits_ref, alpha_ref):
    s_i = si_ref[...][:, None]
    gids = lax.broadcasted_iota(jnp.int32, (1, 64), 1)
    mask = (batch_ref[...][:, None] == gids).astype(jnp.float32)
    masked = jnp.where(mask > 0, s_i, jnp.float32(-1e30))
    seg_max = jnp.max(masked, axis=0, keepdims=True)
    seg_max = jnp.where(seg_max > -1e29, seg_max, 0.0)
    node_max = jnp.sum(mask * seg_max, axis=1, keepdims=True)
    e = jnp.exp(s_i - node_max)
    seg_sum = lax.dot_general(mask, e, (((0,), (0,)), ((), ())),
                              precision=_HIGH)
    node_sum = jnp.dot(mask, seg_sum,
                       preferred_element_type=jnp.float32, precision=_HIGH)
    alpha = e / (node_sum + 1e-16)
    wh = h_ref[...] * alpha
    z_graph = lax.dot_general(mask, wh, (((0,), (0,)), ((), ())),
                              precision=_HIGH)
    logits = _bdot(z_graph, cls_w_ref[...]) + cls_b_ref[...]
    logits_ref[...] = logits
    alpha_ref[...] = alpha


def _tc_pool(s_i, batch1, h_topo, cls_w, cls_b):
    return pl.pallas_call(
        _tc_pool_body,
        out_shape=(jax.ShapeDtypeStruct((64, 1), jnp.float32),
                   jax.ShapeDtypeStruct((N, 1), jnp.float32)),
    )(s_i, batch1, h_topo, cls_w, cls_b)


def kernel(x_sem, edge_index, batch_index, gcn_W, gcn_b, pt_W, pt_b,
           ps_W, ps_b, s1_W, s1_b, s2_W, s2_b, cls_W, cls_b):
    f32 = jnp.float32
    src = edge_index[0].astype(jnp.int32)
    dst = edge_index[1].astype(jnp.int32)
    pad = jnp.full((EPAD - E,), NPAD - 1, jnp.int32)
    src_p = jnp.concatenate([src, pad]).reshape(EPAD // CHUNK, CHUNK)
    dst_p = jnp.concatenate([dst, pad]).reshape(EPAD // CHUNK, CHUNK)
    batch1 = batch_index.astype(jnp.int32)

    ones_deg = jnp.ones((CHUNK, DEGW), f32)
    zeros_deg = jnp.zeros((RPT, DEGW), f32)
    zeros_agg = jnp.zeros((RPT, HHID), f32)

    sc_degree, sc_aggregate = _sc_kernels()
    degp = sc_degree(dst_p, ones_deg, zeros_deg)
    y, dinv = _tc_prescale(x_sem.astype(f32), gcn_W.astype(f32), degp)
    aggp = sc_aggregate(src_p, dst_p, y, zeros_agg)

    h_topo = _tc_proj(aggp, y, dinv, gcn_b.astype(f32).reshape(1, HID))
    z_both, s_i = _tc_score(
        h_topo, x_sem.astype(f32),
        pt_W.astype(f32), pt_b.astype(f32).reshape(1, 32),
        ps_W.astype(f32), ps_b.astype(f32).reshape(1, 32),
        s1_W.astype(f32), s1_b.astype(f32).reshape(1, 16),
        s2_W.astype(f32), s2_b.astype(f32).reshape(1, 1))
    logits2, alpha = _tc_pool(s_i, batch1, h_topo,
                              cls_W.astype(f32),
                              cls_b.astype(f32).reshape(1, 1))
    return (logits2.reshape(64), alpha, z_both[:, 0:32], z_both[:, 32:64])
